# baseline (device time: 1032577 ns/iter reference)
import jax
import jax.numpy as jnp
from jax import lax
from jax.experimental import pallas as pl
from jax.experimental.pallas import tpu as pltpu

C = 512
B = 2 * C
S = 3
L = 2


def kernel(x):
    m, n = x.shape
    steps = m // B

    def body(x_hbm, x_prev, out_ref, send_buf, recvx, recvy,
             copy_sem, x_send_sem, x_recv_sem, y_send_sem, y_recv_sem,
             x_credit, y_credit):
        i = pl.program_id(0)
        my_x = lax.axis_index("x")
        my_y = lax.axis_index("y")
        my_z = lax.axis_index("z")
        xp = (1 - my_x, my_y, my_z)
        yp = (my_x, 1 - my_y, my_z)

        def x_rdma(k):
            slot = k % S
            return pltpu.make_async_remote_copy(
                src_ref=send_buf.at[slot],
                dst_ref=recvx.at[slot],
                send_sem=x_send_sem.at[slot],
                recv_sem=x_recv_sem.at[slot],
                device_id=xp,
                device_id_type=pl.DeviceIdType.MESH,
            )

        def y_rdma(k):
            slot = k % S
            return pltpu.make_async_remote_copy(
                src_ref=recvx.at[slot],
                dst_ref=recvy.at[slot],
                send_sem=y_send_sem.at[slot],
                recv_sem=y_recv_sem.at[slot],
                device_id=yp,
                device_id_type=pl.DeviceIdType.MESH,
            )

        @pl.when(i == 0)
        def _():
            bar = pltpu.get_barrier_semaphore()
            for nbr in (xp, yp):
                pl.semaphore_signal(
                    bar, inc=1,
                    device_id=nbr, device_id_type=pl.DeviceIdType.MESH,
                )
            pl.semaphore_wait(bar, 2)

        @pl.when(i < steps)
        def _():
            @pl.when(i >= S)
            def _():
                pl.semaphore_wait(x_credit, 1)
            cp = pltpu.make_async_copy(
                x_hbm.at[pl.ds(i * B + my_y * C, C), :],
                send_buf.at[i % S],
                copy_sem,
            )
            cp.start()
            cp.wait()
            x_rdma(i).start()

        @pl.when(i >= L)
        def _():
            j = i - L
            slot = j % S
            xr = x_rdma(j)
            xr.wait_recv()

            @pl.when(j >= S)
            def _():
                pl.semaphore_wait(y_credit, 1)
            yr = y_rdma(j)
            yr.start()

            out_ref[pl.ds(my_y * C, C), :] = (
                x_prev[pl.ds(my_y * C, C), :] + recvx[slot])
            xr.wait_send()
            yr.wait_recv()
            out_ref[pl.ds((1 - my_y) * C, C), :] = (
                x_prev[pl.ds((1 - my_y) * C, C), :] + recvy[slot])

            yr.wait_send()

            @pl.when(j <= steps - 1 - S)
            def _():
                pl.semaphore_signal(
                    x_credit, inc=1,
                    device_id=xp, device_id_type=pl.DeviceIdType.MESH,
                )
                pl.semaphore_signal(
                    y_credit, inc=1,
                    device_id=yp, device_id_type=pl.DeviceIdType.MESH,
                )

    lagged = lambda i: (jnp.maximum(i - L, 0), 0)
    return pl.pallas_call(
        body,
        grid=(steps + L,),
        in_specs=[
            pl.BlockSpec(memory_space=pl.ANY),
            pl.BlockSpec((B, n), lagged),
        ],
        out_specs=pl.BlockSpec((B, n), lagged),
        out_shape=jax.ShapeDtypeStruct((m, n), x.dtype),
        scratch_shapes=[
            pltpu.VMEM((S, C, n), x.dtype),
            pltpu.VMEM((S, C, n), x.dtype),
            pltpu.VMEM((S, C, n), x.dtype),
            pltpu.SemaphoreType.DMA,
            pltpu.SemaphoreType.DMA((S,)),
            pltpu.SemaphoreType.DMA((S,)),
            pltpu.SemaphoreType.DMA((S,)),
            pltpu.SemaphoreType.DMA((S,)),
            pltpu.SemaphoreType.REGULAR,
            pltpu.SemaphoreType.REGULAR,
        ],
        compiler_params=pltpu.CompilerParams(
            collective_id=0,
            vmem_limit_bytes=60 * 1024 * 1024,
        ),
    )(x, x)


# device time: 840519 ns/iter; 1.2285x vs baseline; 1.2285x over previous
import jax
import jax.numpy as jnp
from jax import lax
from jax.experimental import pallas as pl
from jax.experimental.pallas import tpu as pltpu

C = 512
B = 2 * C
S = 6
LF = 2
LB = 4


def kernel(x):
    m, n = x.shape
    steps = m // B

    def body(x_hbm, x_prev, out_ref, recvx, recvy,
             x_send_sem, x_recv_sem, y_send_sem, y_recv_sem,
             x_credit, y_credit):
        i = pl.program_id(0)
        my_x = lax.axis_index("x")
        my_y = lax.axis_index("y")
        my_z = lax.axis_index("z")
        xp = (1 - my_x, my_y, my_z)
        yp = (my_x, 1 - my_y, my_z)

        def x_rdma(k):
            slot = k % S
            return pltpu.make_async_remote_copy(
                src_ref=x_hbm.at[pl.ds(k * B + my_y * C, C), :],
                dst_ref=recvx.at[slot],
                send_sem=x_send_sem.at[slot],
                recv_sem=x_recv_sem.at[slot],
                device_id=xp,
                device_id_type=pl.DeviceIdType.MESH,
            )

        def y_rdma(k):
            slot = k % S
            return pltpu.make_async_remote_copy(
                src_ref=recvx.at[slot],
                dst_ref=recvy.at[slot],
                send_sem=y_send_sem.at[slot],
                recv_sem=y_recv_sem.at[slot],
                device_id=yp,
                device_id_type=pl.DeviceIdType.MESH,
            )

        @pl.when(i == 0)
        def _():
            bar = pltpu.get_barrier_semaphore()
            for nbr in (xp, yp):
                pl.semaphore_signal(
                    bar, inc=1,
                    device_id=nbr, device_id_type=pl.DeviceIdType.MESH,
                )
            pl.semaphore_wait(bar, 2)

        @pl.when(i < steps)
        def _():
            @pl.when(i >= S)
            def _():
                pl.semaphore_wait(x_credit, 1)
            x_rdma(i).start()

        @pl.when((i >= LF) & (i < steps + LF))
        def _():
            jf = i - LF
            xr = x_rdma(jf)
            xr.wait_recv()

            @pl.when(jf >= S)
            def _():
                pl.semaphore_wait(y_credit, 1)
            y_rdma(jf).start()

        @pl.when(i >= LB)
        def _():
            j = i - LB
            slot = j % S
            yr = y_rdma(j)
            yr.wait_recv()

            out_ref[pl.ds(my_y * C, C), :] = (
                x_prev[pl.ds(my_y * C, C), :] + recvx[slot])
            out_ref[pl.ds((1 - my_y) * C, C), :] = (
                x_prev[pl.ds((1 - my_y) * C, C), :] + recvy[slot])

            x_rdma(j).wait_send()
            yr.wait_send()

            @pl.when(j <= steps - 1 - S)
            def _():
                pl.semaphore_signal(
                    x_credit, inc=1,
                    device_id=xp, device_id_type=pl.DeviceIdType.MESH,
                )
                pl.semaphore_signal(
                    y_credit, inc=1,
                    device_id=yp, device_id_type=pl.DeviceIdType.MESH,
                )

    lagged = lambda i: (jnp.maximum(i - LB, 0), 0)
    return pl.pallas_call(
        body,
        grid=(steps + LB,),
        in_specs=[
            pl.BlockSpec(memory_space=pl.ANY),
            pl.BlockSpec((B, n), lagged),
        ],
        out_specs=pl.BlockSpec((B, n), lagged),
        out_shape=jax.ShapeDtypeStruct((m, n), x.dtype),
        scratch_shapes=[
            pltpu.VMEM((S, C, n), x.dtype),
            pltpu.VMEM((S, C, n), x.dtype),
            pltpu.SemaphoreType.DMA((S,)),
            pltpu.SemaphoreType.DMA((S,)),
            pltpu.SemaphoreType.DMA((S,)),
            pltpu.SemaphoreType.DMA((S,)),
            pltpu.SemaphoreType.REGULAR,
            pltpu.SemaphoreType.REGULAR,
        ],
        compiler_params=pltpu.CompilerParams(
            collective_id=0,
            vmem_limit_bytes=60 * 1024 * 1024,
        ),
    )(x, x)
